# baseline (speedup 1.0000x reference)
import jax
import jax.numpy as jnp
from jax import lax
from jax.experimental import pallas as pl
from jax.experimental.pallas import tpu as pltpu
from jax.experimental.pallas import tpu_sc as plsc

N = 10000
E = 160000
F = 128
NC = 2
NS = 16
NW = NC * NS
NP = 10240
RPT = NP // NS
K = 100
EPS = E // NS
NCHUNK = EPS // K
K2 = 125
EPW = E // NW
NCHUNK2 = EPW // K2
RB = 640

import functools as _ft


@_ft.lru_cache(maxsize=None)
def _mesh():
    return plsc.VectorSubcoreMesh(core_axis_name="c", subcore_axis_name="s")


def _sc_agg_body(xab_hbm, ids_hbm, zrow_hbm, out_hbm,
                 ib0, ib1, ib2, ib3, rows0, rows1, acc_sh,
                 gs0, gs1, is0, is1, is2, is3):
    c = lax.axis_index("c")
    s = lax.axis_index("s")
    w = c * NS + s
    rbase = s * RPT
    pltpu.sync_copy(zrow_hbm.at[pl.ds(rbase, RPT)],
                    acc_sh.at[pl.ds(rbase, RPT)])

    ib = (ib0, ib1, ib2, ib3)
    isem = (is0, is1, is2, is3)
    rows = (rows0, rows1)
    gsem = (gs0, gs1)

    for j in range(4):
        pltpu.sync_copy(ids_hbm.at[w, j], ib[j])
    for j in range(2):
        pltpu.async_copy(xab_hbm.at[ib[j].at[0]], rows[j], gsem[j])
    plsc.subcore_barrier()

    @pl.loop(0, NCHUNK, step=4)
    def _(g):
        for j in range(4):
            kk = g + j
            b2 = j % 2
            pltpu.make_async_copy(xab_hbm.at[ib[j].at[0]],
                                  rows[b2], gsem[b2]).wait()
            pltpu.sync_copy(rows[b2], acc_sh.at[ib[j].at[1]], add=True)

            @pl.when(kk + 4 < NCHUNK)
            def _():
                pltpu.async_copy(ids_hbm.at[w, kk + 4], ib[j], isem[j])

            @pl.when(kk + 2 < NCHUNK)
            def _():
                j2 = (j + 2) % 4

                @pl.when(kk + 2 >= 4)
                def _():
                    pltpu.make_async_copy(ids_hbm.at[w, kk + 2],
                                          ib[j2], isem[j2]).wait()
                pltpu.async_copy(xab_hbm.at[ib[j2].at[0]], rows[b2], gsem[b2])

    plsc.subcore_barrier()
    pltpu.sync_copy(acc_sh.at[pl.ds(rbase, RPT)],
                    out_hbm.at[pl.ds(c * NP + rbase, RPT)])


@_ft.lru_cache(maxsize=None)
def _sc_agg_k():
  return pl.kernel(
    _sc_agg_body,
    out_type=[jax.ShapeDtypeStruct((2 * NP, F), jnp.float32)],
    mesh=_mesh(),
    scratch_types=[pltpu.VMEM((2, K), jnp.int32),
                   pltpu.VMEM((2, K), jnp.int32),
                   pltpu.VMEM((2, K), jnp.int32),
                   pltpu.VMEM((2, K), jnp.int32),
                   pltpu.VMEM((K, F), jnp.float32),
                   pltpu.VMEM((K, F), jnp.float32),
                   pltpu.VMEM_SHARED((NP, F), jnp.float32),
                   pltpu.SemaphoreType.DMA,
                   pltpu.SemaphoreType.DMA,
                   pltpu.SemaphoreType.DMA,
                   pltpu.SemaphoreType.DMA,
                   pltpu.SemaphoreType.DMA,
                   pltpu.SemaphoreType.DMA],
  )


def _sc_cnt_body(dstw_hbm, zrow_hbm, ones_hbm, out_hbm,
                 dst_all, ones_v, cnt_sh):
    c = lax.axis_index("c")
    s = lax.axis_index("s")
    rbase = s * RPT
    pltpu.sync_copy(ones_hbm, ones_v)
    pltpu.sync_copy(zrow_hbm.at[pl.ds(rbase, RPT)],
                    cnt_sh.at[pl.ds(rbase, RPT)])
    pltpu.sync_copy(dstw_hbm.at[c * NS + s], dst_all)
    plsc.subcore_barrier()

    @pl.loop(0, NCHUNK2)
    def _(kk):
        pltpu.sync_copy(ones_v, cnt_sh.at[dst_all.at[kk]], add=True)

    plsc.subcore_barrier()
    pltpu.sync_copy(cnt_sh.at[pl.ds(rbase, RPT)],
                    out_hbm.at[pl.ds(c * NP + rbase, RPT)])


@_ft.lru_cache(maxsize=None)
def _sc_cnt_k():
  return pl.kernel(
    _sc_cnt_body,
    out_type=[jax.ShapeDtypeStruct((2 * NP, F), jnp.float32)],
    mesh=_mesh(),
    scratch_types=[pltpu.VMEM((NCHUNK2, K2), jnp.int32),
                   pltpu.VMEM((K2, F), jnp.float32),
                   pltpu.VMEM_SHARED((NP, F), jnp.float32)],
  )


def _tc1_body(aggA, aggB, cnt0, cnt1, x, Wl, Wr, b, h1_st):
    inv = 1.0 / jnp.maximum(cnt0[:, :1] + cnt1[:, :1], 1.0)
    wl = Wl[...]
    h = (jnp.dot(aggA[...] * inv, wl[:F], preferred_element_type=jnp.float32)
         + jnp.dot(aggB[...] * inv, wl[F:], preferred_element_type=jnp.float32)
         + jnp.dot(x[...], Wr[...], preferred_element_type=jnp.float32)
         + b[...])
    h1_st[...] = jnp.maximum(h, 0.0)


def _tc1(agg_st, cnt_st, x, Wl, Wr, b):
    nb = NP // RB
    return pl.pallas_call(
        _tc1_body,
        grid=(nb, 2),
        in_specs=[pl.BlockSpec((RB, F), lambda i, j: (i, 0)),
                  pl.BlockSpec((RB, F), lambda i, j: (i + nb, 0)),
                  pl.BlockSpec((RB, F), lambda i, j: (i, 0)),
                  pl.BlockSpec((RB, F), lambda i, j: (i + nb, 0)),
                  pl.BlockSpec((RB, 2 * F), lambda i, j: (i, 0)),
                  pl.BlockSpec((2 * F, F), lambda i, j: (0, j)),
                  pl.BlockSpec((2 * F, F), lambda i, j: (0, j)),
                  pl.BlockSpec((F,), lambda i, j: (j,))],
        out_specs=[pl.BlockSpec((RB, F), lambda i, j: (j * nb + i, 0))],
        out_shape=[jax.ShapeDtypeStruct((2 * NP, F), jnp.float32)],
    )(agg_st, agg_st, cnt_st, cnt_st, x, Wl, Wr, b)[0]


def _tc2_body(aggA, aggB, cnt0, cnt1, hA, hB, Wl, Wr, b,
              Wm0, bm0, Wm1, bm1, out):
    inv = 1.0 / jnp.maximum(cnt0[:, :1] + cnt1[:, :1], 1.0)
    wl = Wl[...]
    wr = Wr[...]
    h2 = (jnp.dot(aggA[...] * inv, wl[:F], preferred_element_type=jnp.float32)
          + jnp.dot(aggB[...] * inv, wl[F:], preferred_element_type=jnp.float32)
          + jnp.dot(hA[...], wr[:F], preferred_element_type=jnp.float32)
          + jnp.dot(hB[...], wr[F:], preferred_element_type=jnp.float32)
          + b[...])
    m = jnp.maximum(jnp.dot(h2, Wm0[...], preferred_element_type=jnp.float32)
                    + bm0[...], 0.0)
    o = jnp.dot(m, Wm1[...], preferred_element_type=jnp.float32) + bm1[...]
    o = o - jnp.max(o, axis=1, keepdims=True)
    e = jnp.exp(o)
    out[...] = e / jnp.sum(e, axis=1, keepdims=True)


def _tc2(agg_st, cnt_st, h1_st, Wl, Wr, b, Wm0, bm0, Wm1, bm1):
    nb = NP // RB
    return pl.pallas_call(
        _tc2_body,
        grid=(nb,),
        in_specs=[pl.BlockSpec((RB, F), lambda i: (i, 0)),
                  pl.BlockSpec((RB, F), lambda i: (i + nb, 0)),
                  pl.BlockSpec((RB, F), lambda i: (i, 0)),
                  pl.BlockSpec((RB, F), lambda i: (i + nb, 0)),
                  pl.BlockSpec((RB, F), lambda i: (i, 0)),
                  pl.BlockSpec((RB, F), lambda i: (i + nb, 0)),
                  pl.BlockSpec((2 * F, 2 * F), lambda i: (0, 0)),
                  pl.BlockSpec((2 * F, 2 * F), lambda i: (0, 0)),
                  pl.BlockSpec((2 * F,), lambda i: (0,)),
                  pl.BlockSpec((2 * F, F), lambda i: (0, 0)),
                  pl.BlockSpec((F,), lambda i: (0,)),
                  pl.BlockSpec((F, 64), lambda i: (0, 0)),
                  pl.BlockSpec((64,), lambda i: (0,))],
        out_specs=[pl.BlockSpec((RB, 64), lambda i: (i, 0))],
        out_shape=[jax.ShapeDtypeStruct((NP, 64), jnp.float32)],
    )(agg_st, agg_st, cnt_st, cnt_st, h1_st, h1_st, Wl, Wr, b,
      Wm0, bm0, Wm1, bm1)[0]


def kernel(x, edge_index, Wl0, Wr0, b0, Wl1, Wr1, b1, Wm0, bm0, Wm1, bm1):
    src = edge_index[0]
    dst = edge_index[1]
    pad = ((0, NP - N), (0, 0))
    xab = jnp.concatenate([jnp.pad(x[:, :F], pad), jnp.pad(x[:, F:], pad)], 0)
    zrow = jnp.zeros((NP, F), jnp.float32)
    ones2 = jnp.ones((K2, F), jnp.float32)
    srcw = jnp.concatenate([src, src + NP]).reshape(NC, NS, NCHUNK, K)
    dstw4 = jnp.broadcast_to(dst.reshape(1, NS, NCHUNK, K),
                             (NC, NS, NCHUNK, K))
    ids = jnp.stack([srcw, dstw4], axis=3).reshape(NW, NCHUNK, 2, K)
    dstw = dst.reshape(NW, NCHUNK2, K2)

    cnt_st = _sc_cnt_k()(dstw, zrow, ones2)[0]
    agg0_st = _sc_agg_k()(xab, ids, zrow)[0]
    h1_st = _tc1(agg0_st, cnt_st, x, Wl0, Wr0, b0)
    agg1_st = _sc_agg_k()(h1_st, ids, zrow)[0]
    out = _tc2(agg1_st, cnt_st, h1_st, Wl1, Wr1, b1, Wm0, bm0, Wm1, bm1)
    return out[:N]


# speedup vs baseline: 1.0192x; 1.0192x over previous
import jax
import jax.numpy as jnp
from jax import lax
from jax.experimental import pallas as pl
from jax.experimental.pallas import tpu as pltpu
from jax.experimental.pallas import tpu_sc as plsc

N = 10000
E = 160000
F = 128
NC = 2
NS = 16
NW = NC * NS
NP = 10240
RPT = NP // NS
K = 125
EPS = E // NS
NCHUNK = EPS // K
K2 = 125
EPW = E // NW
NCHUNK2 = EPW // K2
RB = 640

import functools as _ft


@_ft.lru_cache(maxsize=None)
def _mesh():
    return plsc.VectorSubcoreMesh(core_axis_name="c", subcore_axis_name="s")


def _sc_agg_body(xab_hbm, ids_hbm, zrow_hbm, out_hbm,
                 ib0, ib1, ib2, ib3, rows0, rows1, acc_sh,
                 gs0, gs1, is0, is1, is2, is3):
    c = lax.axis_index("c")
    s = lax.axis_index("s")
    w = c * NS + s
    rbase = s * RPT
    pltpu.sync_copy(zrow_hbm.at[pl.ds(rbase, RPT)],
                    acc_sh.at[pl.ds(rbase, RPT)])

    ib = (ib0, ib1, ib2, ib3)
    isem = (is0, is1, is2, is3)
    rows = (rows0, rows1)
    gsem = (gs0, gs1)

    for j in range(4):
        pltpu.sync_copy(ids_hbm.at[w, j], ib[j])
    for j in range(2):
        pltpu.async_copy(xab_hbm.at[ib[j].at[0]], rows[j], gsem[j])
    plsc.subcore_barrier()

    @pl.loop(0, NCHUNK, step=4)
    def _(g):
        for j in range(4):
            kk = g + j
            b2 = j % 2
            pltpu.make_async_copy(xab_hbm.at[ib[j].at[0]],
                                  rows[b2], gsem[b2]).wait()
            pltpu.sync_copy(rows[b2], acc_sh.at[ib[j].at[1]], add=True)

            @pl.when(kk + 4 < NCHUNK)
            def _():
                pltpu.async_copy(ids_hbm.at[w, kk + 4], ib[j], isem[j])

            @pl.when(kk + 2 < NCHUNK)
            def _():
                j2 = (j + 2) % 4

                @pl.when(kk + 2 >= 4)
                def _():
                    pltpu.make_async_copy(ids_hbm.at[w, kk + 2],
                                          ib[j2], isem[j2]).wait()
                pltpu.async_copy(xab_hbm.at[ib[j2].at[0]], rows[b2], gsem[b2])

    plsc.subcore_barrier()
    pltpu.sync_copy(acc_sh.at[pl.ds(rbase, RPT)],
                    out_hbm.at[pl.ds(c * NP + rbase, RPT)])


@_ft.lru_cache(maxsize=None)
def _sc_agg_k():
  return pl.kernel(
    _sc_agg_body,
    out_type=[jax.ShapeDtypeStruct((2 * NP, F), jnp.float32)],
    mesh=_mesh(),
    scratch_types=[pltpu.VMEM((2, K), jnp.int32),
                   pltpu.VMEM((2, K), jnp.int32),
                   pltpu.VMEM((2, K), jnp.int32),
                   pltpu.VMEM((2, K), jnp.int32),
                   pltpu.VMEM((K, F), jnp.float32),
                   pltpu.VMEM((K, F), jnp.float32),
                   pltpu.VMEM_SHARED((NP, F), jnp.float32),
                   pltpu.SemaphoreType.DMA,
                   pltpu.SemaphoreType.DMA,
                   pltpu.SemaphoreType.DMA,
                   pltpu.SemaphoreType.DMA,
                   pltpu.SemaphoreType.DMA,
                   pltpu.SemaphoreType.DMA],
  )


def _sc_cnt_body(dstw_hbm, zrow_hbm, ones_hbm, out_hbm,
                 dst_all, ones_v, cnt_sh):
    c = lax.axis_index("c")
    s = lax.axis_index("s")
    rbase = s * RPT
    pltpu.sync_copy(ones_hbm, ones_v)
    pltpu.sync_copy(zrow_hbm.at[pl.ds(rbase, RPT)],
                    cnt_sh.at[pl.ds(rbase, RPT)])
    pltpu.sync_copy(dstw_hbm.at[c * NS + s], dst_all)
    plsc.subcore_barrier()

    @pl.loop(0, NCHUNK2)
    def _(kk):
        pltpu.sync_copy(ones_v, cnt_sh.at[dst_all.at[kk]], add=True)

    plsc.subcore_barrier()
    pltpu.sync_copy(cnt_sh.at[pl.ds(rbase, RPT)],
                    out_hbm.at[pl.ds(c * NP + rbase, RPT)])


@_ft.lru_cache(maxsize=None)
def _sc_cnt_k():
  return pl.kernel(
    _sc_cnt_body,
    out_type=[jax.ShapeDtypeStruct((2 * NP, F), jnp.float32)],
    mesh=_mesh(),
    scratch_types=[pltpu.VMEM((NCHUNK2, K2), jnp.int32),
                   pltpu.VMEM((K2, F), jnp.float32),
                   pltpu.VMEM_SHARED((NP, F), jnp.float32)],
  )


def _tc1_body(aggA, aggB, cnt0, cnt1, x, Wl, Wr, b, h1_st):
    inv = 1.0 / jnp.maximum(cnt0[:, :1] + cnt1[:, :1], 1.0)
    wl = Wl[...]
    h = (jnp.dot(aggA[...] * inv, wl[:F], preferred_element_type=jnp.float32)
         + jnp.dot(aggB[...] * inv, wl[F:], preferred_element_type=jnp.float32)
         + jnp.dot(x[...], Wr[...], preferred_element_type=jnp.float32)
         + b[...])
    h1_st[...] = jnp.maximum(h, 0.0)


def _tc1(agg_st, cnt_st, x, Wl, Wr, b):
    nb = NP // RB
    return pl.pallas_call(
        _tc1_body,
        grid=(nb, 2),
        in_specs=[pl.BlockSpec((RB, F), lambda i, j: (i, 0)),
                  pl.BlockSpec((RB, F), lambda i, j: (i + nb, 0)),
                  pl.BlockSpec((RB, F), lambda i, j: (i, 0)),
                  pl.BlockSpec((RB, F), lambda i, j: (i + nb, 0)),
                  pl.BlockSpec((RB, 2 * F), lambda i, j: (i, 0)),
                  pl.BlockSpec((2 * F, F), lambda i, j: (0, j)),
                  pl.BlockSpec((2 * F, F), lambda i, j: (0, j)),
                  pl.BlockSpec((F,), lambda i, j: (j,))],
        out_specs=[pl.BlockSpec((RB, F), lambda i, j: (j * nb + i, 0))],
        out_shape=[jax.ShapeDtypeStruct((2 * NP, F), jnp.float32)],
    )(agg_st, agg_st, cnt_st, cnt_st, x, Wl, Wr, b)[0]


def _tc2_body(aggA, aggB, cnt0, cnt1, hA, hB, Wl, Wr, b,
              Wm0, bm0, Wm1, bm1, out):
    inv = 1.0 / jnp.maximum(cnt0[:, :1] + cnt1[:, :1], 1.0)
    wl = Wl[...]
    wr = Wr[...]
    h2 = (jnp.dot(aggA[...] * inv, wl[:F], preferred_element_type=jnp.float32)
          + jnp.dot(aggB[...] * inv, wl[F:], preferred_element_type=jnp.float32)
          + jnp.dot(hA[...], wr[:F], preferred_element_type=jnp.float32)
          + jnp.dot(hB[...], wr[F:], preferred_element_type=jnp.float32)
          + b[...])
    m = jnp.maximum(jnp.dot(h2, Wm0[...], preferred_element_type=jnp.float32)
                    + bm0[...], 0.0)
    o = jnp.dot(m, Wm1[...], preferred_element_type=jnp.float32) + bm1[...]
    o = o - jnp.max(o, axis=1, keepdims=True)
    e = jnp.exp(o)
    out[...] = e / jnp.sum(e, axis=1, keepdims=True)


def _tc2(agg_st, cnt_st, h1_st, Wl, Wr, b, Wm0, bm0, Wm1, bm1):
    nb = NP // RB
    return pl.pallas_call(
        _tc2_body,
        grid=(nb,),
        in_specs=[pl.BlockSpec((RB, F), lambda i: (i, 0)),
                  pl.BlockSpec((RB, F), lambda i: (i + nb, 0)),
                  pl.BlockSpec((RB, F), lambda i: (i, 0)),
                  pl.BlockSpec((RB, F), lambda i: (i + nb, 0)),
                  pl.BlockSpec((RB, F), lambda i: (i, 0)),
                  pl.BlockSpec((RB, F), lambda i: (i + nb, 0)),
                  pl.BlockSpec((2 * F, 2 * F), lambda i: (0, 0)),
                  pl.BlockSpec((2 * F, 2 * F), lambda i: (0, 0)),
                  pl.BlockSpec((2 * F,), lambda i: (0,)),
                  pl.BlockSpec((2 * F, F), lambda i: (0, 0)),
                  pl.BlockSpec((F,), lambda i: (0,)),
                  pl.BlockSpec((F, 64), lambda i: (0, 0)),
                  pl.BlockSpec((64,), lambda i: (0,))],
        out_specs=[pl.BlockSpec((RB, 64), lambda i: (i, 0))],
        out_shape=[jax.ShapeDtypeStruct((N, 64), jnp.float32)],
    )(agg_st, agg_st, cnt_st, cnt_st, h1_st, h1_st, Wl, Wr, b,
      Wm0, bm0, Wm1, bm1)[0]


def kernel(x, edge_index, Wl0, Wr0, b0, Wl1, Wr1, b1, Wm0, bm0, Wm1, bm1):
    src = edge_index[0]
    dst = edge_index[1]
    pad = ((0, NP - N), (0, 0))
    xab = jnp.concatenate([jnp.pad(x[:, :F], pad), jnp.pad(x[:, F:], pad)], 0)
    zrow = jnp.zeros((NP, F), jnp.float32)
    ones2 = jnp.ones((K2, F), jnp.float32)
    srcw = jnp.concatenate([src, src + NP]).reshape(NC, NS, NCHUNK, K)
    dstw4 = jnp.broadcast_to(dst.reshape(1, NS, NCHUNK, K),
                             (NC, NS, NCHUNK, K))
    ids = jnp.stack([srcw, dstw4], axis=3).reshape(NW, NCHUNK, 2, K)
    dstw = dst.reshape(NW, NCHUNK2, K2)

    cnt_st = _sc_cnt_k()(dstw, zrow, ones2)[0]
    agg0_st = _sc_agg_k()(xab, ids, zrow)[0]
    h1_st = _tc1(agg0_st, cnt_st, x, Wl0, Wr0, b0)
    agg1_st = _sc_agg_k()(h1_st, ids, zrow)[0]
    return _tc2(agg1_st, cnt_st, h1_st, Wl1, Wr1, b1, Wm0, bm0, Wm1, bm1)
